# Initial kernel scaffold; baseline (speedup 1.0000x reference)
import jax
import jax.numpy as jnp
from jax.experimental import pallas as pl


def kernel(adj, image_emb, text_emb, W_img, b_img, W_txt, b_txt, modal_weight, item_id_emb, user_emb, image_original_adj, text_original_adj):
    raise NotImplementedError("write your pallas kernel here")


# speedup vs baseline: 9.8110x; 9.8110x over previous
import functools
import jax
import jax.numpy as jnp
from jax.experimental import pallas as pl
from jax.experimental.pallas import tpu as pltpu

N = 4096
NT = 8192
D = 64
K = 10

TILE_T = 128
TILE_H = 256
TILE_U = 512


def _proj_body(img_ref, wimg_ref, bimg_ref, txt_ref, wtxt_ref, btxt_ref,
               xi_ref, xt_ref):
    fi = jnp.dot(img_ref[...], wimg_ref[...],
                 preferred_element_type=jnp.float32) + bimg_ref[...]
    ft = jnp.dot(txt_ref[...], wtxt_ref[...],
                 preferred_element_type=jnp.float32) + btxt_ref[...]
    ni = jnp.sqrt(jnp.sum(fi * fi, axis=1, keepdims=True))
    nt = jnp.sqrt(jnp.sum(ft * ft, axis=1, keepdims=True))
    xi_ref[...] = fi / ni
    xt_ref[...] = ft / nt


def _topk_mask(s):
    cur = s
    out = jnp.zeros_like(s)
    for _ in range(K):
        m = jnp.max(cur, axis=1, keepdims=True)
        hit = cur == m
        out = jnp.where(hit, cur, out)
        cur = jnp.where(hit, -jnp.inf, cur)
    return out


def _simtopk_body(w_ref, xi_tile_ref, xi_full_ref, xt_tile_ref, xt_full_ref,
                  a_ref, r_ref):
    si = jnp.dot(xi_tile_ref[...], xi_full_ref[...].T,
                 preferred_element_type=jnp.float32)
    a_ref[...] = w_ref[0] * _topk_mask(si)
    st = jnp.dot(xt_tile_ref[...], xt_full_ref[...].T,
                 preferred_element_type=jnp.float32)
    a = a_ref[...] + w_ref[1] * _topk_mask(st)
    a_ref[...] = a
    r_ref[...] = jnp.sum(a, axis=1, keepdims=True)


def _degree_body(r_ref, e_ref, d_ref, ed_ref):
    p = jnp.power(r_ref[...], -0.5)
    d = jnp.where(jnp.isinf(p), 0.0, p)
    d_ref[...] = d
    ed_ref[...] = d * e_ref[...]


def _itemadj_body(w_ref, a_ref, d_ref, ed_ref, e_ref, io_ref, to_ref, hn_ref):
    learned = d_ref[...] * jnp.dot(a_ref[...], ed_ref[...],
                                   preferred_element_type=jnp.float32)
    orig = jnp.dot(w_ref[0] * io_ref[...] + w_ref[1] * to_ref[...], e_ref[...],
                   preferred_element_type=jnp.float32)
    h = 0.1 * learned + 0.9 * orig
    nrm = jnp.maximum(jnp.sqrt(jnp.sum(h * h, axis=1, keepdims=True)), 1e-12)
    hn_ref[...] = h / nrm


def _mm_body(adj_ref, x_ref, o_ref):
    o_ref[...] = jnp.dot(adj_ref[...], x_ref[...],
                         preferred_element_type=jnp.float32)


def _final_body(adj_ref, e1_ref, e1t_ref, ego_ref, hn_ref, o_ref):
    e2 = jnp.dot(adj_ref[...], e1_ref[...], preferred_element_type=jnp.float32)
    mean = (ego_ref[...] + e1t_ref[...] + e2) * (1.0 / 3.0)
    j = pl.program_id(0)
    flag = jnp.where(j >= (N // TILE_U), 1.0, 0.0)
    o_ref[...] = mean + flag * hn_ref[...]


def kernel(adj, image_emb, text_emb, W_img, b_img, W_txt, b_txt, modal_weight,
           item_id_emb, user_emb, image_original_adj, text_original_adj):
    f32 = jnp.float32
    w = jax.nn.softmax(modal_weight, axis=0)

    xi, xt = pl.pallas_call(
        _proj_body,
        grid=(4,),
        in_specs=[
            pl.BlockSpec((N // 4, 1024), lambda i: (i, 0)),
            pl.BlockSpec((1024, D), lambda i: (0, 0)),
            pl.BlockSpec((1, D), lambda i: (0, 0)),
            pl.BlockSpec((N // 4, 384), lambda i: (i, 0)),
            pl.BlockSpec((384, D), lambda i: (0, 0)),
            pl.BlockSpec((1, D), lambda i: (0, 0)),
        ],
        out_specs=[
            pl.BlockSpec((N // 4, D), lambda i: (i, 0)),
            pl.BlockSpec((N // 4, D), lambda i: (i, 0)),
        ],
        out_shape=[
            jax.ShapeDtypeStruct((N, D), f32),
            jax.ShapeDtypeStruct((N, D), f32),
        ],
    )(image_emb, W_img, b_img.reshape(1, D), text_emb, W_txt,
      b_txt.reshape(1, D))

    n_t = N // TILE_T
    a_sp, rowsum = pl.pallas_call(
        _simtopk_body,
        grid=(n_t,),
        in_specs=[
            pl.BlockSpec(memory_space=pltpu.SMEM),
            pl.BlockSpec((TILE_T, D), lambda i: (i, 0)),
            pl.BlockSpec((N, D), lambda i: (0, 0)),
            pl.BlockSpec((TILE_T, D), lambda i: (i, 0)),
            pl.BlockSpec((N, D), lambda i: (0, 0)),
        ],
        out_specs=[
            pl.BlockSpec((TILE_T, N), lambda i: (i, 0)),
            pl.BlockSpec((TILE_T, 1), lambda i: (i, 0)),
        ],
        out_shape=[
            jax.ShapeDtypeStruct((N, N), f32),
            jax.ShapeDtypeStruct((N, 1), f32),
        ],
    )(w, xi, xi, xt, xt)

    d_vec, ed = pl.pallas_call(
        _degree_body,
        in_specs=[
            pl.BlockSpec((N, 1), lambda: (0, 0)),
            pl.BlockSpec((N, D), lambda: (0, 0)),
        ],
        out_specs=[
            pl.BlockSpec((N, 1), lambda: (0, 0)),
            pl.BlockSpec((N, D), lambda: (0, 0)),
        ],
        out_shape=[
            jax.ShapeDtypeStruct((N, 1), f32),
            jax.ShapeDtypeStruct((N, D), f32),
        ],
    )(rowsum, item_id_emb)

    n_h = N // TILE_H
    h_norm = pl.pallas_call(
        _itemadj_body,
        grid=(n_h,),
        in_specs=[
            pl.BlockSpec(memory_space=pltpu.SMEM),
            pl.BlockSpec((TILE_H, N), lambda i: (i, 0)),
            pl.BlockSpec((TILE_H, 1), lambda i: (i, 0)),
            pl.BlockSpec((N, D), lambda i: (0, 0)),
            pl.BlockSpec((N, D), lambda i: (0, 0)),
            pl.BlockSpec((TILE_H, N), lambda i: (i, 0)),
            pl.BlockSpec((TILE_H, N), lambda i: (i, 0)),
        ],
        out_specs=pl.BlockSpec((TILE_H, D), lambda i: (i, 0)),
        out_shape=jax.ShapeDtypeStruct((N, D), f32),
    )(w, a_sp, d_vec, ed, item_id_emb, image_original_adj, text_original_adj)

    ego0 = jnp.concatenate([user_emb, item_id_emb], axis=0)
    n_u = NT // TILE_U
    e1 = pl.pallas_call(
        _mm_body,
        grid=(n_u,),
        in_specs=[
            pl.BlockSpec((TILE_U, NT), lambda i: (i, 0)),
            pl.BlockSpec((NT, D), lambda i: (0, 0)),
        ],
        out_specs=pl.BlockSpec((TILE_U, D), lambda i: (i, 0)),
        out_shape=jax.ShapeDtypeStruct((NT, D), f32),
    )(adj, ego0)

    out = pl.pallas_call(
        _final_body,
        grid=(n_u,),
        in_specs=[
            pl.BlockSpec((TILE_U, NT), lambda i: (i, 0)),
            pl.BlockSpec((NT, D), lambda i: (0, 0)),
            pl.BlockSpec((TILE_U, D), lambda i: (i, 0)),
            pl.BlockSpec((TILE_U, D), lambda i: (i, 0)),
            pl.BlockSpec((TILE_U, D),
                         lambda i: (jnp.maximum(i - N // TILE_U, 0), 0)),
        ],
        out_specs=pl.BlockSpec((TILE_U, D), lambda i: (i, 0)),
        out_shape=jax.ShapeDtypeStruct((NT, D), f32),
    )(adj, e1, e1, ego0, h_norm)

    return out[:N], out[N:]


# speedup vs baseline: 12.2484x; 1.2484x over previous
import functools
import jax
import jax.numpy as jnp
from jax.experimental import pallas as pl
from jax.experimental.pallas import tpu as pltpu

N = 4096
NT = 8192
D = 64
K = 10

TILE_T = 128
TILE_H = 256
TILE_U = 512


def _proj_body(img_ref, wimg_ref, bimg_ref, txt_ref, wtxt_ref, btxt_ref,
               xi_ref, xt_ref):
    fi = jnp.dot(img_ref[...], wimg_ref[...],
                 preferred_element_type=jnp.float32) + bimg_ref[...]
    ft = jnp.dot(txt_ref[...], wtxt_ref[...],
                 preferred_element_type=jnp.float32) + btxt_ref[...]
    ni = jnp.sqrt(jnp.sum(fi * fi, axis=1, keepdims=True))
    nt = jnp.sqrt(jnp.sum(ft * ft, axis=1, keepdims=True))
    xi_ref[...] = fi / ni
    xt_ref[...] = ft / nt


def _topk_mask(s):
    cur = s
    m = None
    for _ in range(K):
        m = jnp.max(cur, axis=1, keepdims=True)
        cur = jnp.where(cur == m, -jnp.inf, cur)
    return jnp.where(s >= m, s, 0.0)


def _simtopk_body(w_ref, xi_tile_ref, xi_full_ref, xt_tile_ref, xt_full_ref,
                  a_ref, r_ref):
    si = jnp.dot(xi_tile_ref[...], xi_full_ref[...].T,
                 preferred_element_type=jnp.float32)
    ai = w_ref[0] * _topk_mask(si)
    st = jnp.dot(xt_tile_ref[...], xt_full_ref[...].T,
                 preferred_element_type=jnp.float32)
    a = ai + w_ref[1] * _topk_mask(st)
    a_ref[...] = a.astype(jnp.bfloat16)
    r_ref[...] = jnp.sum(a, axis=1, keepdims=True)


def _degree_body(r_ref, e_ref, d_ref, ed_ref):
    p = jnp.power(r_ref[...], -0.5)
    d = jnp.where(jnp.isinf(p), 0.0, p)
    d_ref[...] = d
    ed_ref[...] = d * e_ref[...]


def _itemadj_body(w_ref, a_ref, d_ref, ed_ref, e_ref, io_ref, to_ref, hn_ref):
    learned = d_ref[...] * jnp.dot(
        a_ref[...], ed_ref[...].astype(jnp.bfloat16),
        preferred_element_type=jnp.float32)
    orig = jnp.dot(w_ref[0] * io_ref[...] + w_ref[1] * to_ref[...], e_ref[...],
                   preferred_element_type=jnp.float32)
    h = 0.1 * learned + 0.9 * orig
    nrm = jnp.maximum(jnp.sqrt(jnp.sum(h * h, axis=1, keepdims=True)), 1e-12)
    hn_ref[...] = h / nrm


def _mm_body(adj_ref, x_ref, o_ref):
    o_ref[...] = jnp.dot(adj_ref[...], x_ref[...],
                         preferred_element_type=jnp.float32)


def _final_body(adj_ref, e1_ref, e1t_ref, ego_ref, hn_ref, o_ref):
    e2 = jnp.dot(adj_ref[...], e1_ref[...], preferred_element_type=jnp.float32)
    mean = (ego_ref[...] + e1t_ref[...] + e2) * (1.0 / 3.0)
    j = pl.program_id(0)
    flag = jnp.where(j >= (N // TILE_U), 1.0, 0.0)
    o_ref[...] = mean + flag * hn_ref[...]


def kernel(adj, image_emb, text_emb, W_img, b_img, W_txt, b_txt, modal_weight,
           item_id_emb, user_emb, image_original_adj, text_original_adj):
    f32 = jnp.float32
    w = jax.nn.softmax(modal_weight, axis=0)

    xi, xt = pl.pallas_call(
        _proj_body,
        grid=(4,),
        in_specs=[
            pl.BlockSpec((N // 4, 1024), lambda i: (i, 0)),
            pl.BlockSpec((1024, D), lambda i: (0, 0)),
            pl.BlockSpec((1, D), lambda i: (0, 0)),
            pl.BlockSpec((N // 4, 384), lambda i: (i, 0)),
            pl.BlockSpec((384, D), lambda i: (0, 0)),
            pl.BlockSpec((1, D), lambda i: (0, 0)),
        ],
        out_specs=[
            pl.BlockSpec((N // 4, D), lambda i: (i, 0)),
            pl.BlockSpec((N // 4, D), lambda i: (i, 0)),
        ],
        out_shape=[
            jax.ShapeDtypeStruct((N, D), f32),
            jax.ShapeDtypeStruct((N, D), f32),
        ],
    )(image_emb, W_img, b_img.reshape(1, D), text_emb, W_txt,
      b_txt.reshape(1, D))

    n_t = N // TILE_T
    a_sp, rowsum = pl.pallas_call(
        _simtopk_body,
        grid=(n_t,),
        in_specs=[
            pl.BlockSpec(memory_space=pltpu.SMEM),
            pl.BlockSpec((TILE_T, D), lambda i: (i, 0)),
            pl.BlockSpec((N, D), lambda i: (0, 0)),
            pl.BlockSpec((TILE_T, D), lambda i: (i, 0)),
            pl.BlockSpec((N, D), lambda i: (0, 0)),
        ],
        out_specs=[
            pl.BlockSpec((TILE_T, N), lambda i: (i, 0)),
            pl.BlockSpec((TILE_T, 1), lambda i: (i, 0)),
        ],
        out_shape=[
            jax.ShapeDtypeStruct((N, N), jnp.bfloat16),
            jax.ShapeDtypeStruct((N, 1), f32),
        ],
    )(w, xi, xi, xt, xt)

    d_vec, ed = pl.pallas_call(
        _degree_body,
        in_specs=[
            pl.BlockSpec((N, 1), lambda: (0, 0)),
            pl.BlockSpec((N, D), lambda: (0, 0)),
        ],
        out_specs=[
            pl.BlockSpec((N, 1), lambda: (0, 0)),
            pl.BlockSpec((N, D), lambda: (0, 0)),
        ],
        out_shape=[
            jax.ShapeDtypeStruct((N, 1), f32),
            jax.ShapeDtypeStruct((N, D), f32),
        ],
    )(rowsum, item_id_emb)

    n_h = N // TILE_H
    h_norm = pl.pallas_call(
        _itemadj_body,
        grid=(n_h,),
        in_specs=[
            pl.BlockSpec(memory_space=pltpu.SMEM),
            pl.BlockSpec((TILE_H, N), lambda i: (i, 0)),
            pl.BlockSpec((TILE_H, 1), lambda i: (i, 0)),
            pl.BlockSpec((N, D), lambda i: (0, 0)),
            pl.BlockSpec((N, D), lambda i: (0, 0)),
            pl.BlockSpec((TILE_H, N), lambda i: (i, 0)),
            pl.BlockSpec((TILE_H, N), lambda i: (i, 0)),
        ],
        out_specs=pl.BlockSpec((TILE_H, D), lambda i: (i, 0)),
        out_shape=jax.ShapeDtypeStruct((N, D), f32),
    )(w, a_sp, d_vec, ed, item_id_emb, image_original_adj, text_original_adj)

    ego0 = jnp.concatenate([user_emb, item_id_emb], axis=0)
    n_u = NT // TILE_U
    e1 = pl.pallas_call(
        _mm_body,
        grid=(n_u,),
        in_specs=[
            pl.BlockSpec((TILE_U, NT), lambda i: (i, 0)),
            pl.BlockSpec((NT, D), lambda i: (0, 0)),
        ],
        out_specs=pl.BlockSpec((TILE_U, D), lambda i: (i, 0)),
        out_shape=jax.ShapeDtypeStruct((NT, D), f32),
    )(adj, ego0)

    out = pl.pallas_call(
        _final_body,
        grid=(n_u,),
        in_specs=[
            pl.BlockSpec((TILE_U, NT), lambda i: (i, 0)),
            pl.BlockSpec((NT, D), lambda i: (0, 0)),
            pl.BlockSpec((TILE_U, D), lambda i: (i, 0)),
            pl.BlockSpec((TILE_U, D), lambda i: (i, 0)),
            pl.BlockSpec((TILE_U, D),
                         lambda i: (jnp.maximum(i - N // TILE_U, 0), 0)),
        ],
        out_specs=pl.BlockSpec((TILE_U, D), lambda i: (i, 0)),
        out_shape=jax.ShapeDtypeStruct((NT, D), f32),
    )(adj, e1, e1, ego0, h_norm)

    return out[:N], out[N:]


# speedup vs baseline: 13.2741x; 1.0837x over previous
import functools
import jax
import jax.numpy as jnp
from jax.experimental import pallas as pl
from jax.experimental.pallas import tpu as pltpu

N = 4096
NT = 8192
D = 64
K = 10

TILE_T = 128
TILE_H = 256
TILE_U = 512


def _proj_body(img_ref, wimg_ref, bimg_ref, txt_ref, wtxt_ref, btxt_ref,
               xi_ref, xt_ref):
    fi = jnp.dot(img_ref[...], wimg_ref[...],
                 preferred_element_type=jnp.float32) + bimg_ref[...]
    ft = jnp.dot(txt_ref[...], wtxt_ref[...],
                 preferred_element_type=jnp.float32) + btxt_ref[...]
    ni = jnp.sqrt(jnp.sum(fi * fi, axis=1, keepdims=True))
    nt = jnp.sqrt(jnp.sum(ft * ft, axis=1, keepdims=True))
    xi_ref[...] = fi / ni
    xt_ref[...] = ft / nt


def _topk_mask(s):
    sb = s.astype(jnp.bfloat16)
    cur = sb
    m = None
    neg = jnp.asarray(-jnp.inf, jnp.bfloat16)
    for _ in range(K):
        m = jnp.max(cur, axis=1, keepdims=True)
        cur = jnp.where(cur == m, neg, cur)
    return jnp.where(sb >= m, s, 0.0)


def _simtopk_body(w_ref, xi_tile_ref, xi_full_ref, xt_tile_ref, xt_full_ref,
                  a_ref, r_ref):
    si = jnp.dot(xi_tile_ref[...], xi_full_ref[...].T,
                 preferred_element_type=jnp.float32)
    ai = w_ref[0] * _topk_mask(si)
    st = jnp.dot(xt_tile_ref[...], xt_full_ref[...].T,
                 preferred_element_type=jnp.float32)
    a = ai + w_ref[1] * _topk_mask(st)
    a_ref[...] = a.astype(jnp.bfloat16)
    r_ref[...] = jnp.sum(a, axis=1, keepdims=True)


def _degree_body(r_ref, e_ref, d_ref, ed_ref):
    p = jnp.power(r_ref[...], -0.5)
    d = jnp.where(jnp.isinf(p), 0.0, p)
    d_ref[...] = d
    ed_ref[...] = d * e_ref[...]


def _itemadj_body(w_ref, a_ref, d_ref, ed_ref, e_ref, io_ref, to_ref, hn_ref):
    learned = d_ref[...] * jnp.dot(
        a_ref[...], ed_ref[...].astype(jnp.bfloat16),
        preferred_element_type=jnp.float32)
    orig = jnp.dot(w_ref[0] * io_ref[...] + w_ref[1] * to_ref[...], e_ref[...],
                   preferred_element_type=jnp.float32)
    h = 0.1 * learned + 0.9 * orig
    nrm = jnp.maximum(jnp.sqrt(jnp.sum(h * h, axis=1, keepdims=True)), 1e-12)
    hn_ref[...] = h / nrm


def _mm_body(adj_ref, x_ref, o_ref):
    o_ref[...] = jnp.dot(adj_ref[...], x_ref[...],
                         preferred_element_type=jnp.float32)


def _final_body(adj_ref, e1_ref, e1t_ref, ego_ref, hn_ref, o_ref):
    e2 = jnp.dot(adj_ref[...], e1_ref[...], preferred_element_type=jnp.float32)
    mean = (ego_ref[...] + e1t_ref[...] + e2) * (1.0 / 3.0)
    j = pl.program_id(0)
    flag = jnp.where(j >= (N // TILE_U), 1.0, 0.0)
    o_ref[...] = mean + flag * hn_ref[...]


def kernel(adj, image_emb, text_emb, W_img, b_img, W_txt, b_txt, modal_weight,
           item_id_emb, user_emb, image_original_adj, text_original_adj):
    f32 = jnp.float32
    w = jax.nn.softmax(modal_weight, axis=0)

    xi, xt = pl.pallas_call(
        _proj_body,
        grid=(4,),
        in_specs=[
            pl.BlockSpec((N // 4, 1024), lambda i: (i, 0)),
            pl.BlockSpec((1024, D), lambda i: (0, 0)),
            pl.BlockSpec((1, D), lambda i: (0, 0)),
            pl.BlockSpec((N // 4, 384), lambda i: (i, 0)),
            pl.BlockSpec((384, D), lambda i: (0, 0)),
            pl.BlockSpec((1, D), lambda i: (0, 0)),
        ],
        out_specs=[
            pl.BlockSpec((N // 4, D), lambda i: (i, 0)),
            pl.BlockSpec((N // 4, D), lambda i: (i, 0)),
        ],
        out_shape=[
            jax.ShapeDtypeStruct((N, D), f32),
            jax.ShapeDtypeStruct((N, D), f32),
        ],
    )(image_emb, W_img, b_img.reshape(1, D), text_emb, W_txt,
      b_txt.reshape(1, D))

    n_t = N // TILE_T
    a_sp, rowsum = pl.pallas_call(
        _simtopk_body,
        grid=(n_t,),
        in_specs=[
            pl.BlockSpec(memory_space=pltpu.SMEM),
            pl.BlockSpec((TILE_T, D), lambda i: (i, 0)),
            pl.BlockSpec((N, D), lambda i: (0, 0)),
            pl.BlockSpec((TILE_T, D), lambda i: (i, 0)),
            pl.BlockSpec((N, D), lambda i: (0, 0)),
        ],
        out_specs=[
            pl.BlockSpec((TILE_T, N), lambda i: (i, 0)),
            pl.BlockSpec((TILE_T, 1), lambda i: (i, 0)),
        ],
        out_shape=[
            jax.ShapeDtypeStruct((N, N), jnp.bfloat16),
            jax.ShapeDtypeStruct((N, 1), f32),
        ],
    )(w, xi, xi, xt, xt)

    d_vec, ed = pl.pallas_call(
        _degree_body,
        in_specs=[
            pl.BlockSpec((N, 1), lambda: (0, 0)),
            pl.BlockSpec((N, D), lambda: (0, 0)),
        ],
        out_specs=[
            pl.BlockSpec((N, 1), lambda: (0, 0)),
            pl.BlockSpec((N, D), lambda: (0, 0)),
        ],
        out_shape=[
            jax.ShapeDtypeStruct((N, 1), f32),
            jax.ShapeDtypeStruct((N, D), f32),
        ],
    )(rowsum, item_id_emb)

    n_h = N // TILE_H
    h_norm = pl.pallas_call(
        _itemadj_body,
        grid=(n_h,),
        in_specs=[
            pl.BlockSpec(memory_space=pltpu.SMEM),
            pl.BlockSpec((TILE_H, N), lambda i: (i, 0)),
            pl.BlockSpec((TILE_H, 1), lambda i: (i, 0)),
            pl.BlockSpec((N, D), lambda i: (0, 0)),
            pl.BlockSpec((N, D), lambda i: (0, 0)),
            pl.BlockSpec((TILE_H, N), lambda i: (i, 0)),
            pl.BlockSpec((TILE_H, N), lambda i: (i, 0)),
        ],
        out_specs=pl.BlockSpec((TILE_H, D), lambda i: (i, 0)),
        out_shape=jax.ShapeDtypeStruct((N, D), f32),
    )(w, a_sp, d_vec, ed, item_id_emb, image_original_adj, text_original_adj)

    ego0 = jnp.concatenate([user_emb, item_id_emb], axis=0)
    n_u = NT // TILE_U
    e1 = pl.pallas_call(
        _mm_body,
        grid=(n_u,),
        in_specs=[
            pl.BlockSpec((TILE_U, NT), lambda i: (i, 0)),
            pl.BlockSpec((NT, D), lambda i: (0, 0)),
        ],
        out_specs=pl.BlockSpec((TILE_U, D), lambda i: (i, 0)),
        out_shape=jax.ShapeDtypeStruct((NT, D), f32),
    )(adj, ego0)

    out = pl.pallas_call(
        _final_body,
        grid=(n_u,),
        in_specs=[
            pl.BlockSpec((TILE_U, NT), lambda i: (i, 0)),
            pl.BlockSpec((NT, D), lambda i: (0, 0)),
            pl.BlockSpec((TILE_U, D), lambda i: (i, 0)),
            pl.BlockSpec((TILE_U, D), lambda i: (i, 0)),
            pl.BlockSpec((TILE_U, D),
                         lambda i: (jnp.maximum(i - N // TILE_U, 0), 0)),
        ],
        out_specs=pl.BlockSpec((TILE_U, D), lambda i: (i, 0)),
        out_shape=jax.ShapeDtypeStruct((NT, D), f32),
    )(adj, e1, e1, ego0, h_norm)

    return out[:N], out[N:]
